# Initial kernel scaffold; baseline (speedup 1.0000x reference)
import functools

import jax
import jax.numpy as jnp
from jax import lax
from jax.experimental import pallas as pl
from jax.experimental.pallas import tpu as pltpu
from jax.experimental.pallas import tpu_sc as plsc

_HIDDEN = 64
_EPS = 1e-12
_C = 80


def _rsqrt(x):
    i = lax.bitcast_convert_type(x, jnp.int32)
    i = jnp.int32(0x5F3759DF) - lax.shift_right_arithmetic(i, 1)
    y = lax.bitcast_convert_type(i, jnp.float32)
    for _ in range(3):
        y = y * (1.5 - 0.5 * x * y * y)
    return y


def _make_kernel(n_tok, n_hdr):
    info = plsc.get_sparse_core_info()
    nw = info.num_cores * info.num_subcores
    tok_per_w = n_tok // nw
    hdr_per_w = n_hdr // nw
    assert n_tok % (nw * _C) == 0 and n_hdr % (nw * _C) == 0

    mesh = plsc.VectorSubcoreMesh(core_axis_name="c", subcore_axis_name="s")

    @functools.partial(
        pl.kernel,
        mesh=mesh,
        out_type=(
            jax.ShapeDtypeStruct((n_tok, _HIDDEN), jnp.float32),
            jax.ShapeDtypeStruct((n_hdr, _HIDDEN), jnp.float32),
        ),
        scratch_types=[
            pltpu.VMEM((_C,), jnp.int32),
            pltpu.VMEM((_C,), jnp.int32),
            pltpu.VMEM((_C,), jnp.int32),
            pltpu.VMEM((_C, _HIDDEN), jnp.float32),
            pltpu.VMEM((_C, _HIDDEN), jnp.float32),
            pltpu.VMEM((_C, _HIDDEN), jnp.float32),
            pltpu.VMEM((_HIDDEN,), jnp.float32),
            pltpu.VMEM((_HIDDEN,), jnp.float32),
            pltpu.SemaphoreType.DMA,
            pltpu.SemaphoreType.DMA,
            pltpu.SemaphoreType.DMA,
        ],
    )
    def k(tok_hbm, pos_hbm, typ_hbm, hdr_hbm, hty_hbm,
          word_t, header_t, pos_t, type_t, lnw_hbm, lnb_hbm,
          out_tok, out_hdr,
          idx0, idx1, idx2, buf0, buf1, buf2, wv, bvv,
          sem0, sem1, sem2):
        wid = lax.axis_index("s") * info.num_cores + lax.axis_index("c")
        pltpu.sync_copy(lnw_hbm, wv)
        pltpu.sync_copy(lnb_hbm, bvv)

        def ln_rows(n_tables):
            def row(r, _):
                vs = []
                for g in range(_HIDDEN // 16):
                    v = buf0[r, pl.ds(g * 16, 16)] + buf1[r, pl.ds(g * 16, 16)]
                    if n_tables == 3:
                        v = v + buf2[r, pl.ds(g * 16, 16)]
                    vs.append(v)
                total = jnp.sum((vs[0] + vs[1]) + (vs[2] + vs[3]), axis=0)
                m = total * (1.0 / _HIDDEN)
                ds = [v - m for v in vs]
                sq = (ds[0] * ds[0] + ds[1] * ds[1]) + (ds[2] * ds[2] + ds[3] * ds[3])
                var = jnp.sum(sq, axis=0) * (1.0 / _HIDDEN)
                rstd = _rsqrt(jnp.full((16,), var + _EPS, jnp.float32))
                for g in range(_HIDDEN // 16):
                    o = ds[g] * rstd * wv[pl.ds(g * 16, 16)] + bvv[pl.ds(g * 16, 16)]
                    buf0[r, pl.ds(g * 16, 16)] = o
                return 0
            lax.fori_loop(0, _C, row, 0)

        def path(n_tables, iA, iB, iC, tA, tB, tC, out_hbm, per_w):
            base = wid * per_w

            def chunk(ci, _):
                off = base + ci * _C
                pltpu.sync_copy(iA.at[pl.ds(off, _C)], idx0)
                pltpu.sync_copy(iB.at[pl.ds(off, _C)], idx1)
                if n_tables == 3:
                    pltpu.sync_copy(iC.at[pl.ds(off, _C)], idx2)
                c0 = pltpu.async_copy(tA.at[idx0], buf0, sem0)
                c1 = pltpu.async_copy(tB.at[idx1], buf1, sem1)
                if n_tables == 3:
                    c2 = pltpu.async_copy(tC.at[idx2], buf2, sem2)
                c0.wait()
                c1.wait()
                if n_tables == 3:
                    c2.wait()
                ln_rows(n_tables)
                pltpu.sync_copy(buf0, out_hbm.at[pl.ds(off, _C)])
                return 0

            lax.fori_loop(0, per_w // _C, chunk, 0)

        path(3, tok_hbm, pos_hbm, typ_hbm, word_t, pos_t, type_t,
             out_tok, tok_per_w)
        path(2, hdr_hbm, hty_hbm, None, header_t, type_t, None,
             out_hdr, hdr_per_w)

    return k


def kernel(input_tok, input_tok_type, input_tok_pos, input_header,
           input_header_type, word_table, header_table, pos_table,
           type_table, ln_weight, ln_bias):
    b, t = input_tok.shape
    _, h = input_header.shape
    n_tok, n_hdr = b * t, b * h
    k = _make_kernel(n_tok, n_hdr)
    out_tok, out_hdr = k(
        input_tok.reshape(-1).astype(jnp.int32),
        input_tok_pos.reshape(-1).astype(jnp.int32),
        input_tok_type.reshape(-1).astype(jnp.int32),
        input_header.reshape(-1).astype(jnp.int32),
        input_header_type.reshape(-1).astype(jnp.int32),
        word_table, header_table, pos_table, type_table,
        ln_weight, ln_bias,
    )
    return (out_tok.reshape(b, t, _HIDDEN), out_hdr.reshape(b, h, _HIDDEN))


# speedup vs baseline: 1.0083x; 1.0083x over previous
import functools

import jax
import jax.numpy as jnp
from jax import lax
from jax.experimental import pallas as pl
from jax.experimental.pallas import tpu as pltpu
from jax.experimental.pallas import tpu_sc as plsc

_HIDDEN = 64
_EPS = 1e-12
_C = 80


_GATHER_DNUMS = lax.GatherDimensionNumbers(
    offset_dims=(), collapsed_slice_dims=(0,), start_index_map=(0,))


def _permute(v, idx):
    return lax.gather(v, idx[:, None], _GATHER_DNUMS, slice_sizes=(1,),
                      mode=lax.GatherScatterMode.PROMISE_IN_BOUNDS)


def _hsum(v):
    lanes = lax.iota(jnp.int32, 16)
    for s in (8, 4, 2, 1):
        v = v + _permute(v, lanes ^ s)
    return v


def _rsqrt(x):
    i = lax.bitcast_convert_type(x, jnp.int32)
    i = jnp.int32(0x5F3759DF) - lax.shift_right_arithmetic(i, 1)
    y = lax.bitcast_convert_type(i, jnp.float32)
    for _ in range(3):
        y = y * (1.5 - 0.5 * x * y * y)
    return y


def _make_kernel(n_tok, n_hdr):
    info = plsc.get_sparse_core_info()
    nw = info.num_cores * info.num_subcores
    tok_per_w = n_tok // nw
    hdr_per_w = n_hdr // nw
    assert n_tok % (nw * _C) == 0 and n_hdr % (nw * _C) == 0

    mesh = plsc.VectorSubcoreMesh(core_axis_name="c", subcore_axis_name="s")

    @functools.partial(
        pl.kernel,
        mesh=mesh,
        compiler_params=pltpu.CompilerParams(use_tc_tiling_on_sc=False),
        out_type=(
            jax.ShapeDtypeStruct((n_tok, _HIDDEN), jnp.float32),
            jax.ShapeDtypeStruct((n_hdr, _HIDDEN), jnp.float32),
        ),
        scratch_types=[
            pltpu.VMEM((_C,), jnp.int32),
            pltpu.VMEM((_C,), jnp.int32),
            pltpu.VMEM((_C,), jnp.int32),
            pltpu.VMEM((_C, _HIDDEN), jnp.float32),
            pltpu.VMEM((_C, _HIDDEN), jnp.float32),
            pltpu.VMEM((_C, _HIDDEN), jnp.float32),
            pltpu.VMEM((_HIDDEN,), jnp.float32),
            pltpu.VMEM((_HIDDEN,), jnp.float32),
            pltpu.SemaphoreType.DMA,
            pltpu.SemaphoreType.DMA,
            pltpu.SemaphoreType.DMA,
        ],
    )
    def k(tok_hbm, pos_hbm, typ_hbm, hdr_hbm, hty_hbm,
          word_t, header_t, pos_t, type_t, lnw_hbm, lnb_hbm,
          out_tok, out_hdr,
          idx0, idx1, idx2, buf0, buf1, buf2, wv, bvv,
          sem0, sem1, sem2):
        wid = lax.axis_index("s") * info.num_cores + lax.axis_index("c")
        pltpu.sync_copy(lnw_hbm, wv)
        pltpu.sync_copy(lnb_hbm, bvv)

        def ln_rows(n_tables):
            def row(r, _):
                vs = []
                for g in range(_HIDDEN // 16):
                    v = buf0[r, pl.ds(g * 16, 16)] + buf1[r, pl.ds(g * 16, 16)]
                    if n_tables == 3:
                        v = v + buf2[r, pl.ds(g * 16, 16)]
                    vs.append(v)
                total = _hsum((vs[0] + vs[1]) + (vs[2] + vs[3]))
                m = total * (1.0 / _HIDDEN)
                ds = [v - m for v in vs]
                sq = (ds[0] * ds[0] + ds[1] * ds[1]) + (ds[2] * ds[2] + ds[3] * ds[3])
                var = _hsum(sq) * (1.0 / _HIDDEN)
                rstd = _rsqrt(var + _EPS)
                for g in range(_HIDDEN // 16):
                    o = ds[g] * rstd * wv[pl.ds(g * 16, 16)] + bvv[pl.ds(g * 16, 16)]
                    buf0[r, pl.ds(g * 16, 16)] = o
                return 0
            lax.fori_loop(0, _C, row, 0)

        def path(n_tables, iA, iB, iC, tA, tB, tC, out_hbm, per_w):
            base = wid * per_w

            def chunk(ci, _):
                off = base + ci * _C
                pltpu.sync_copy(iA.at[pl.ds(off, _C)], idx0)
                pltpu.sync_copy(iB.at[pl.ds(off, _C)], idx1)
                if n_tables == 3:
                    pltpu.sync_copy(iC.at[pl.ds(off, _C)], idx2)
                c0 = pltpu.async_copy(tA.at[idx0], buf0, sem0)
                c1 = pltpu.async_copy(tB.at[idx1], buf1, sem1)
                if n_tables == 3:
                    c2 = pltpu.async_copy(tC.at[idx2], buf2, sem2)
                c0.wait()
                c1.wait()
                if n_tables == 3:
                    c2.wait()
                ln_rows(n_tables)
                pltpu.sync_copy(buf0, out_hbm.at[pl.ds(off, _C)])
                return 0

            lax.fori_loop(0, per_w // _C, chunk, 0)

        path(3, tok_hbm, pos_hbm, typ_hbm, word_t, pos_t, type_t,
             out_tok, tok_per_w)
        path(2, hdr_hbm, hty_hbm, None, header_t, type_t, None,
             out_hdr, hdr_per_w)

    return k


def kernel(input_tok, input_tok_type, input_tok_pos, input_header,
           input_header_type, word_table, header_table, pos_table,
           type_table, ln_weight, ln_bias):
    b, t = input_tok.shape
    _, h = input_header.shape
    n_tok, n_hdr = b * t, b * h
    k = _make_kernel(n_tok, n_hdr)
    out_tok, out_hdr = k(
        input_tok.reshape(-1).astype(jnp.int32),
        input_tok_pos.reshape(-1).astype(jnp.int32),
        input_tok_type.reshape(-1).astype(jnp.int32),
        input_header.reshape(-1).astype(jnp.int32),
        input_header_type.reshape(-1).astype(jnp.int32),
        word_table, header_table, pos_table, type_table,
        ln_weight, ln_bias,
    )
    return (out_tok.reshape(b, t, _HIDDEN), out_hdr.reshape(b, h, _HIDDEN))
